# Initial kernel scaffold; baseline (speedup 1.0000x reference)
import jax
import jax.numpy as jnp
from jax.experimental import pallas as pl


def kernel(coord, feat, offset, reference_index, fc1_w, fc3_w, lin1_w, lin1_b, lin2_w, lin2_b, n1_g, n1_b, n2_g, n2_b, n3_g, n3_b, m1_g, m1_b, m2_g, m2_b):
    raise NotImplementedError("write your pallas kernel here")


# speedup vs baseline: 2.1994x; 2.1994x over previous
import functools

import jax
import jax.numpy as jnp
from jax import lax
from jax.experimental import pallas as pl
from jax.experimental.pallas import tpu as pltpu
from jax.experimental.pallas import tpu_sc as plsc

N, K, C = 10000, 32, 128
NC, NS = 2, 16
NW = NC * NS
P = 320
NP = NW * P
CP = 4
RPC = CP * K
NCH = P // CP
IDX_PER_W = (N * K) // NW
G16 = C // 16
EPS = 1e-5


def _wid():
    return lax.axis_index("s") * NC + lax.axis_index("c")


@functools.lru_cache(maxsize=None)
def _build_sc_kernels():
    mesh = plsc.VectorSubcoreMesh(core_axis_name="c", subcore_axis_name="s")
    cparams = pltpu.CompilerParams(needs_layout_passes=False)

    @functools.partial(
        pl.kernel,
        out_type=jax.ShapeDtypeStruct((NW, N), jnp.float32),
        mesh=mesh,
        compiler_params=cparams,
        scratch_types=[
            pltpu.VMEM((IDX_PER_W,), jnp.int32),
            pltpu.VMEM((N,), jnp.float32),
        ],
    )
    def bincount(idx_hbm, pc_hbm, idx_v, cnt_v):
        w = _wid()
        pltpu.sync_copy(idx_hbm.at[w], idx_v)
        zeros = jnp.zeros((16,), jnp.float32)

        def zbody(i, c):
            cnt_v[pl.ds(i * 16, 16)] = zeros
            return c

        lax.fori_loop(0, N // 16, zbody, 0)
        ones = jnp.ones((16,), jnp.float32)

        def abody(i, c):
            v = idx_v[pl.ds(i * 16, 16)]
            plsc.addupdate_scatter(cnt_v, [v], ones)
            return c

        lax.fori_loop(0, IDX_PER_W // 16, abody, 0)
        pltpu.sync_copy(cnt_v, pc_hbm.at[w])

    @functools.partial(
        pl.kernel,
        out_type=jax.ShapeDtypeStruct((NP, C), jnp.float32),
        mesh=mesh,
        compiler_params=cparams,
        scratch_types=[
            pltpu.VMEM((NCH, RPC), jnp.int32),
            pltpu.VMEM((RPC, C), jnp.float32),
            pltpu.VMEM((RPC, C), jnp.float32),
            pltpu.VMEM((P, C), jnp.float32),
            pltpu.SemaphoreType.DMA,
            pltpu.SemaphoreType.DMA,
        ],
    )
    def gather_max(table_hbm, idx_hbm, out_hbm, idx_v, buf0, buf1, out_v, sem0, sem1):
        w = _wid()
        pltpu.sync_copy(idx_hbm.at[w], idx_v)
        pltpu.async_copy(table_hbm.at[idx_v.at[0]], buf0, sem0)

        def compute(buf, ch):
            def pbody(p, c):
                base = p * K
                for g in range(G16):
                    sl = pl.ds(g * 16, 16)
                    accs = [buf[base + r0, sl] for r0 in range(4)]
                    for r in range(4, K):
                        accs[r % 4] = jnp.maximum(accs[r % 4], buf[base + r, sl])
                    acc = jnp.maximum(
                        jnp.maximum(accs[0], accs[1]),
                        jnp.maximum(accs[2], accs[3]),
                    )
                    out_v[ch * CP + p, sl] = acc
                return c

            lax.fori_loop(0, CP, pbody, 0)

        def jbody(j, c):
            ch0 = 2 * j
            ch1 = 2 * j + 1
            pltpu.make_async_copy(table_hbm.at[idx_v.at[ch0]], buf0, sem0).wait()
            pltpu.async_copy(table_hbm.at[idx_v.at[ch1]], buf1, sem1)
            compute(buf0, ch0)

            @pl.when(j < NCH // 2 - 1)
            def _():
                pltpu.async_copy(table_hbm.at[idx_v.at[ch0 + 2]], buf0, sem0)

            pltpu.make_async_copy(table_hbm.at[idx_v.at[ch1]], buf1, sem1).wait()
            compute(buf1, ch1)
            return c

        lax.fori_loop(0, NCH // 2, jbody, 0)
        pltpu.sync_copy(out_v, out_hbm.at[pl.ds(w * P, P)])

    return bincount, gather_max


def _mlp_body(pc_ref, feat_ref, fc1_ref, l1w_ref, l1b_ref, l2w_ref, l2b_ref,
              n1g_ref, n1b_ref, m1g_ref, m1b_ref, m2g_ref, m2b_ref, out_ref):
    f = feat_ref[...]
    y0 = jnp.dot(f, fc1_ref[...], preferred_element_type=jnp.float32)
    m0 = jnp.mean(y0, axis=0, keepdims=True)
    d0 = y0 - m0
    v0 = jnp.mean(d0 * d0, axis=0, keepdims=True)
    x1 = jnp.maximum(d0 * lax.rsqrt(v0 + EPS) * n1g_ref[...] + n1b_ref[...], 0.0)

    wrow = jnp.sum(pc_ref[...], axis=0, keepdims=True) * (1.0 / (N * K))

    y1 = jnp.dot(x1, l1w_ref[...], preferred_element_type=jnp.float32) + l1b_ref[...]
    mw1 = jnp.dot(wrow, y1, preferred_element_type=jnp.float32)
    d1 = y1 - mw1
    vw1 = jnp.dot(wrow, d1 * d1, preferred_element_type=jnp.float32)
    h1 = jnp.maximum(d1 * lax.rsqrt(vw1 + EPS) * m1g_ref[...] + m1b_ref[...], 0.0)

    y2 = jnp.dot(h1, l2w_ref[...], preferred_element_type=jnp.float32) + l2b_ref[...]
    mw2 = jnp.dot(wrow, y2, preferred_element_type=jnp.float32)
    d2 = y2 - mw2
    vw2 = jnp.dot(wrow, d2 * d2, preferred_element_type=jnp.float32)
    out_ref[...] = jnp.maximum(
        d2 * lax.rsqrt(vw2 + EPS) * m2g_ref[...] + m2b_ref[...], 0.0)


def _out_body(pool_ref, feat_ref, fc3_ref, n2g_ref, n2b_ref, n3g_ref, n3b_ref,
              out_ref):
    x = pool_ref[...][:N]
    m2 = jnp.mean(x, axis=0, keepdims=True)
    dd = x - m2
    v2 = jnp.mean(dd * dd, axis=0, keepdims=True)
    xn = jnp.maximum(dd * lax.rsqrt(v2 + EPS) * n2g_ref[...] + n2b_ref[...], 0.0)
    y = jnp.dot(xn, fc3_ref[...], preferred_element_type=jnp.float32)
    m3 = jnp.mean(y, axis=0, keepdims=True)
    d3 = y - m3
    v3 = jnp.mean(d3 * d3, axis=0, keepdims=True)
    yn = d3 * lax.rsqrt(v3 + EPS) * n3g_ref[...] + n3b_ref[...]
    out_ref[...] = jnp.maximum(feat_ref[...] + yn, 0.0)


def kernel(coord, feat, offset, reference_index, fc1_w, fc3_w, lin1_w, lin1_b,
           lin2_w, lin2_b, n1_g, n1_b, n2_g, n2_b, n3_g, n3_b, m1_g, m1_b,
           m2_g, m2_b):
    bincount, gather_max = _build_sc_kernels()

    idx2d = reference_index.reshape(NW, IDX_PER_W)
    pc = bincount(idx2d)

    r1 = lambda a: a.reshape(1, C)
    h2 = pl.pallas_call(
        _mlp_body,
        out_shape=jax.ShapeDtypeStruct((N, C), jnp.float32),
    )(pc, feat, fc1_w.T, lin1_w.T, r1(lin1_b), lin2_w.T, r1(lin2_b),
      r1(n1_g), r1(n1_b), r1(m1_g), r1(m1_b), r1(m2_g), r1(m2_b))

    idx_pad = jnp.pad(reference_index, ((0, NP - N), (0, 0)))
    idx3 = idx_pad.reshape(NW, NCH, RPC)
    pooled = gather_max(h2, idx3)

    out = pl.pallas_call(
        _out_body,
        out_shape=jax.ShapeDtypeStruct((N, C), jnp.float32),
    )(pooled, feat, fc3_w.T, r1(n2_g), r1(n2_b), r1(n3_g), r1(n3_b))

    return (coord, out, offset)


# speedup vs baseline: 6.9789x; 3.1732x over previous
import functools

import jax
import jax.numpy as jnp
from jax import lax
from jax.experimental import pallas as pl
from jax.experimental.pallas import tpu as pltpu
from jax.experimental.pallas import tpu_sc as plsc

N, K, C = 10000, 32, 128
NC, NS = 2, 16
NW = NC * NS
CW = C // NW
CHP = 400
NCHK = N // CHP
PB = CHP // 16
IDX_PER_W = (N * K) // NW
EPS = 1e-5


def _wid():
    return lax.axis_index("s") * NC + lax.axis_index("c")


@functools.lru_cache(maxsize=None)
def _build_sc_kernels():
    mesh = plsc.VectorSubcoreMesh(core_axis_name="c", subcore_axis_name="s")
    cparams = pltpu.CompilerParams(needs_layout_passes=False)

    @functools.partial(
        pl.kernel,
        out_type=jax.ShapeDtypeStruct((NW, N), jnp.float32),
        mesh=mesh,
        compiler_params=cparams,
        scratch_types=[
            pltpu.VMEM((IDX_PER_W,), jnp.int32),
            pltpu.VMEM((N,), jnp.float32),
        ],
    )
    def bincount(idx_hbm, pc_hbm, idx_v, cnt_v):
        w = _wid()
        pltpu.sync_copy(idx_hbm.at[w], idx_v)
        zeros = jnp.zeros((16,), jnp.float32)

        def zbody(i, c):
            cnt_v[pl.ds(i * 16, 16)] = zeros
            return c

        lax.fori_loop(0, N // 16, zbody, 0)
        ones = jnp.ones((16,), jnp.float32)

        def abody(i, c):
            v = idx_v[pl.ds(i * 16, 16)]
            plsc.addupdate_scatter(cnt_v, [v], ones)
            return c

        lax.fori_loop(0, IDX_PER_W // 16, abody, 0)
        pltpu.sync_copy(cnt_v, pc_hbm.at[w])

    @functools.partial(
        pl.kernel,
        out_type=jax.ShapeDtypeStruct((C, N), jnp.float32),
        mesh=mesh,
        compiler_params=cparams,
        scratch_types=[
            pltpu.VMEM((CW, N), jnp.float32),
            pltpu.VMEM((K, CHP), jnp.int32),
            pltpu.VMEM((K, CHP), jnp.int32),
            pltpu.VMEM((CW, N), jnp.float32),
            pltpu.SemaphoreType.DMA,
            pltpu.SemaphoreType.DMA,
        ],
    )
    def gather_max(tblT_hbm, idxc_hbm, outT_hbm, tbl_v, idx0, idx1, out_v,
                   sem0, sem1):
        w = _wid()
        pltpu.sync_copy(tblT_hbm.at[pl.ds(w * CW, CW)], tbl_v)
        pltpu.async_copy(idxc_hbm.at[0], idx0, sem0)
        cvecs = [jnp.full((16,), c, jnp.int32) for c in range(CW)]

        def compute(idx_v, ch):
            n0 = ch * CHP

            def pbbody(pb, c):
                col = pb * 16
                v0 = idx_v[0, pl.ds(col, 16)]
                accs = [plsc.load_gather(tbl_v, [cvecs[c4], v0])
                        for c4 in range(CW)]
                for k in range(1, K):
                    vk = idx_v[k, pl.ds(col, 16)]
                    for c4 in range(CW):
                        accs[c4] = jnp.maximum(
                            accs[c4], plsc.load_gather(tbl_v, [cvecs[c4], vk]))
                for c4 in range(CW):
                    out_v[c4, pl.ds(n0 + col, 16)] = accs[c4]
                return c

            lax.fori_loop(0, PB, pbbody, 0)

        def jbody(j, c):
            ch0 = 2 * j
            ch1 = 2 * j + 1
            pltpu.make_async_copy(idxc_hbm.at[ch0], idx0, sem0).wait()
            pltpu.async_copy(idxc_hbm.at[ch1], idx1, sem1)
            compute(idx0, ch0)
            pltpu.async_copy(idxc_hbm.at[ch0 + 2], idx0, sem0)
            pltpu.make_async_copy(idxc_hbm.at[ch1], idx1, sem1).wait()
            compute(idx1, ch1)
            return c

        lax.fori_loop(0, NCHK // 2, jbody, 0)
        pltpu.make_async_copy(idxc_hbm.at[NCHK - 1], idx0, sem0).wait()
        compute(idx0, NCHK - 1)
        pltpu.sync_copy(out_v, outT_hbm.at[pl.ds(w * CW, CW)])

    return bincount, gather_max


def _mlp_body(pc_ref, feat_ref, fc1_ref, l1w_ref, l1b_ref, l2w_ref, l2b_ref,
              n1g_ref, n1b_ref, m1g_ref, m1b_ref, m2g_ref, m2b_ref, out_ref):
    f = feat_ref[...]
    y0 = jnp.dot(f, fc1_ref[...], preferred_element_type=jnp.float32)
    m0 = jnp.mean(y0, axis=0, keepdims=True)
    d0 = y0 - m0
    v0 = jnp.mean(d0 * d0, axis=0, keepdims=True)
    x1 = jnp.maximum(d0 * lax.rsqrt(v0 + EPS) * n1g_ref[...] + n1b_ref[...], 0.0)

    wrow = jnp.sum(pc_ref[...], axis=0, keepdims=True) * (1.0 / (N * K))

    y1 = jnp.dot(x1, l1w_ref[...], preferred_element_type=jnp.float32) + l1b_ref[...]
    mw1 = jnp.dot(wrow, y1, preferred_element_type=jnp.float32)
    d1 = y1 - mw1
    vw1 = jnp.dot(wrow, d1 * d1, preferred_element_type=jnp.float32)
    h1 = jnp.maximum(d1 * lax.rsqrt(vw1 + EPS) * m1g_ref[...] + m1b_ref[...], 0.0)

    y2 = jnp.dot(h1, l2w_ref[...], preferred_element_type=jnp.float32) + l2b_ref[...]
    mw2 = jnp.dot(wrow, y2, preferred_element_type=jnp.float32)
    d2 = y2 - mw2
    vw2 = jnp.dot(wrow, d2 * d2, preferred_element_type=jnp.float32)
    h2 = jnp.maximum(d2 * lax.rsqrt(vw2 + EPS) * m2g_ref[...] + m2b_ref[...], 0.0)
    out_ref[...] = h2.T


def _out_body(pool_ref, feat_ref, fc3_ref, n2g_ref, n2b_ref, n3g_ref, n3b_ref,
              out_ref):
    x = pool_ref[...].T
    m2 = jnp.mean(x, axis=0, keepdims=True)
    dd = x - m2
    v2 = jnp.mean(dd * dd, axis=0, keepdims=True)
    xn = jnp.maximum(dd * lax.rsqrt(v2 + EPS) * n2g_ref[...] + n2b_ref[...], 0.0)
    y = jnp.dot(xn, fc3_ref[...], preferred_element_type=jnp.float32)
    m3 = jnp.mean(y, axis=0, keepdims=True)
    d3 = y - m3
    v3 = jnp.mean(d3 * d3, axis=0, keepdims=True)
    yn = d3 * lax.rsqrt(v3 + EPS) * n3g_ref[...] + n3b_ref[...]
    out_ref[...] = jnp.maximum(feat_ref[...] + yn, 0.0)


def kernel(coord, feat, offset, reference_index, fc1_w, fc3_w, lin1_w, lin1_b,
           lin2_w, lin2_b, n1_g, n1_b, n2_g, n2_b, n3_g, n3_b, m1_g, m1_b,
           m2_g, m2_b):
    bincount, gather_max = _build_sc_kernels()

    idx2d = reference_index.reshape(NW, IDX_PER_W)
    pc = bincount(idx2d)

    r1 = lambda a: a.reshape(1, C)
    h2t = pl.pallas_call(
        _mlp_body,
        out_shape=jax.ShapeDtypeStruct((C, N), jnp.float32),
    )(pc, feat, fc1_w.T, lin1_w.T, r1(lin1_b), lin2_w.T, r1(lin2_b),
      r1(n1_g), r1(n1_b), r1(m1_g), r1(m1_b), r1(m2_g), r1(m2_b))

    idxc = reference_index.T.reshape(K, NCHK, CHP).transpose(1, 0, 2)
    pooled = gather_max(h2t, idxc)

    out = pl.pallas_call(
        _out_body,
        out_shape=jax.ShapeDtypeStruct((N, C), jnp.float32),
    )(pooled, feat, fc3_w.T, r1(n2_g), r1(n2_b), r1(n3_g), r1(n3_b))

    return (coord, out, offset)
